# baseline (speedup 1.0000x reference)
import functools

import jax
import jax.numpy as jnp
from jax import lax
from jax.experimental import pallas as pl
from jax.experimental.pallas import tpu as pltpu
from jax.experimental.pallas import tpu_sc as plsc

_N_NODES = 2048
_N_EDGES = 32768
_D_E = 16
_N_CORES = 1
_N_WORKERS = 16 * _N_CORES
_WIN = _N_NODES // _N_WORKERS
_CH = 128
_G = _N_EDGES // 16

_mesh = plsc.VectorSubcoreMesh(
    core_axis_name="c", subcore_axis_name="s", num_cores=_N_CORES)


@functools.partial(
    pl.kernel,
    out_type=jax.ShapeDtypeStruct((_N_NODES, _D_E), jnp.float32),
    mesh=_mesh,
    compiler_params=pltpu.CompilerParams(needs_layout_passes=False),
    scratch_types=[
        pltpu.VMEM((_N_EDGES,), jnp.int32),
        pltpu.VMEM((_CH, _D_E), jnp.float32),
        pltpu.VMEM((_WIN, _D_E), jnp.float32),
    ],
)
def _seg_sum(e_hbm, recv_hbm, out_hbm, r_full, e_v, acc):
    c = lax.axis_index("c")
    s = lax.axis_index("s")
    wid = c * 16 + s
    lo = wid * _WIN
    hi = lo + _WIN

    pltpu.sync_copy(recv_hbm, r_full)

    def first_group_ge(thresh):
        def step(_, lh):
            low, high = lh
            mid = jnp.minimum((low + high) // 2, _G - 1)
            x = r_full[pl.ds(mid * 16, 16)]
            p = x[15] >= thresh
            low = lax.select(p, low, mid + 1)
            high = lax.select(p, mid, high)
            return low, high

        low, _ = lax.fori_loop(0, 12, step, (0, _G))
        return low

    gs = first_group_ge(lo)
    ge = first_group_ge(hi)

    zrow = jnp.zeros((16,), jnp.float32)

    def zero_body(j, _):
        acc[j] = zrow
        return 0

    lax.fori_loop(0, _WIN, zero_body, 0)

    col = lax.iota(jnp.int32, 16)
    c0 = gs // 8
    c1 = jnp.minimum(ge // 8 + 1, _N_EDGES // _CH)
    c1 = lax.select(gs >= _G, c0, c1)

    def chunk_body(k, _):
        pltpu.sync_copy(e_hbm.at[pl.ds(k * _CH, _CH)], e_v)
        for j in range(_CH):
            ivec = jnp.full((16,), k * _CH + j, jnp.int32)
            rsplat = plsc.load_gather(r_full, [ivec])
            mask = (rsplat >= lo) & (rsplat < hi)
            plsc.addupdate_scatter(acc, [rsplat - lo, col], e_v[j], mask=mask)
        return 0

    lax.fori_loop(c0, c1, chunk_body, 0)

    pltpu.sync_copy(acc, out_hbm.at[pl.ds(lo, _WIN)])


def _combine_body(h_ref, wl_ref, esum_ref, wr_ref, u_ref, out_ref):
    gl = lax.dot_general(
        h_ref[...], wl_ref[...], (((1,), (1,)), ((), ())),
        preferred_element_type=jnp.float32,
    )
    agg = lax.dot_general(
        esum_ref[...], wr_ref[...], (((1,), (1,)), ((), ())),
        preferred_element_type=jnp.float32,
    )
    out_ref[...] = jnp.concatenate([gl, agg, u_ref[...]], axis=1)


def kernel(h, e, receivers, u, W_l, W_r, W_attn):
    del W_attn
    n_nodes = h.shape[0]
    recv = receivers.astype(jnp.int32)
    esum = _seg_sum(e, recv)
    out = pl.pallas_call(
        _combine_body,
        out_shape=jax.ShapeDtypeStruct(
            (n_nodes, W_l.shape[0] + W_r.shape[0] + u.shape[1]), jnp.float32),
    )(h, W_l, esum, W_r, u)
    return out


# speedup vs baseline: 1.4722x; 1.4722x over previous
import functools

import jax
import jax.numpy as jnp
from jax import lax
from jax.experimental import pallas as pl
from jax.experimental.pallas import tpu as pltpu
from jax.experimental.pallas import tpu_sc as plsc

_N_NODES = 2048
_N_EDGES = 32768
_D_E = 16
_N_CORES = 2
_N_WORKERS = 16 * _N_CORES
_WIN = _N_NODES // _N_WORKERS
_CH = 128
_G = _N_EDGES // 16

_mesh = plsc.VectorSubcoreMesh(
    core_axis_name="c", subcore_axis_name="s", num_cores=_N_CORES)


@functools.partial(
    pl.kernel,
    out_type=jax.ShapeDtypeStruct((_N_NODES, _D_E), jnp.float32),
    mesh=_mesh,
    compiler_params=pltpu.CompilerParams(needs_layout_passes=False),
    scratch_types=[
        pltpu.VMEM((_N_EDGES,), jnp.int32),
        pltpu.VMEM((_CH, _D_E), jnp.float32),
        pltpu.VMEM((_CH, _D_E), jnp.float32),
        pltpu.VMEM((_WIN, _D_E), jnp.float32),
        pltpu.SemaphoreType.DMA,
        pltpu.SemaphoreType.DMA,
    ],
)
def _seg_sum(e_hbm, recv_hbm, out_hbm, r_full, e_v0, e_v1, acc, sem0, sem1):
    c = lax.axis_index("c")
    s = lax.axis_index("s")
    wid = c * 16 + s
    lo = wid * _WIN
    hi = lo + _WIN

    pltpu.sync_copy(recv_hbm, r_full)

    def first_group_ge(thresh):
        def step(_, lh):
            low, high = lh
            mid = jnp.minimum((low + high) // 2, _G - 1)
            x = r_full[pl.ds(mid * 16, 16)]
            p = x[15] >= thresh
            low = lax.select(p, low, mid + 1)
            high = lax.select(p, mid, high)
            return low, high

        low, _ = lax.fori_loop(0, 12, step, (0, _G))
        return low

    gs = first_group_ge(lo)
    ge = first_group_ge(hi)

    zrow = jnp.zeros((16,), jnp.float32)

    def zero_body(j, _):
        acc[j] = zrow
        return 0

    lax.fori_loop(0, _WIN, zero_body, 0)

    col = lax.iota(jnp.int32, 16)
    gpc = _CH // 16
    c0 = gs // gpc
    c1 = jnp.minimum(ge // gpc + 1, _N_EDGES // _CH)
    c1 = lax.select(gs >= _G, c0, c1)

    def _start(k, buf, sem):
        pltpu.async_copy(e_hbm.at[pl.ds(k * _CH, _CH)], buf, sem)

    def _process(k, buf, sem):
        pltpu.make_async_copy(e_hbm.at[pl.ds(k * _CH, _CH)], buf, sem).wait()
        base = jnp.full((16,), k * _CH, jnp.int32)
        for j0 in range(0, _CH, 4):
            rs = [plsc.load_gather(r_full, [base + (j0 + t)]) for t in range(4)]
            rows = [buf[j0 + t] for t in range(4)]
            for t in range(4):
                mask = (rs[t] >= lo) & (rs[t] < hi)
                plsc.addupdate_scatter(acc, [rs[t] - lo, col], rows[t],
                                       mask=mask)

    @pl.when(c0 < c1)
    def _():
        _start(c0, e_v0, sem0)

    def chunk_body(k, _):
        even = ((k - c0) % 2) == 0

        @pl.when(k + 1 < c1)
        def _():
            @pl.when(even)
            def _():
                _start(k + 1, e_v1, sem1)

            @pl.when(jnp.logical_not(even))
            def _():
                _start(k + 1, e_v0, sem0)

        @pl.when(even)
        def _():
            _process(k, e_v0, sem0)

        @pl.when(jnp.logical_not(even))
        def _():
            _process(k, e_v1, sem1)

        return 0

    lax.fori_loop(c0, c1, chunk_body, 0)

    pltpu.sync_copy(acc, out_hbm.at[pl.ds(lo, _WIN)])


def _combine_body(h_ref, wl_ref, esum_ref, wr_ref, u_ref, out_ref):
    gl = lax.dot_general(
        h_ref[...], wl_ref[...], (((1,), (1,)), ((), ())),
        preferred_element_type=jnp.float32,
    )
    agg = lax.dot_general(
        esum_ref[...], wr_ref[...], (((1,), (1,)), ((), ())),
        preferred_element_type=jnp.float32,
    )
    out_ref[...] = jnp.concatenate([gl, agg, u_ref[...]], axis=1)


def kernel(h, e, receivers, u, W_l, W_r, W_attn):
    del W_attn
    n_nodes = h.shape[0]
    recv = receivers.astype(jnp.int32)
    esum = _seg_sum(e, recv)
    out = pl.pallas_call(
        _combine_body,
        out_shape=jax.ShapeDtypeStruct(
            (n_nodes, W_l.shape[0] + W_r.shape[0] + u.shape[1]), jnp.float32),
    )(h, W_l, esum, W_r, u)
    return out
